# baseline (speedup 1.0000x reference)
import functools

import jax
import jax.numpy as jnp
from jax import lax
from jax.experimental import pallas as pl
from jax.experimental.pallas import tpu as pltpu
from jax.experimental.pallas import tpu_sc as plsc

_B, _T, _C = 16, 1024, 768
_NC, _NS = 2, 16
_NW = _NC * _NS
_ROWS_PER_W = _B * _T // _NW
_CHUNK = 32
_NCH = _ROWS_PER_W // _CHUNK
_NBUF = 4
_LANES = 16


def _shuffle_body(x_hbm, idx_hbm, out_hbm, idx_v, *rest):
    bufs = rest[:_NBUF]
    gsems = rest[_NBUF:2 * _NBUF]
    ssems = rest[2 * _NBUF:]
    b = lax.axis_index("s")
    half = lax.axis_index("c")
    out_base = (b * _NC + half) * _ROWS_PER_W

    pltpu.sync_copy(idx_hbm.at[pl.ds(half * _NCH, _NCH)], idx_v)
    boff = (b * _T).astype(jnp.int32)
    for j in range(_NCH):
        for i in range(_CHUNK // _LANES):
            sl = pl.ds(i * _LANES, _LANES)
            idx_v[j, sl] = idx_v[j, sl] + boff

    gs = [None] * _NCH
    ss = [None] * _NCH
    for j in range(_NBUF - 1):
        gs[j] = pltpu.async_copy(x_hbm.at[idx_v.at[j]], bufs[j], gsems[j])
    for j in range(_NCH):
        nx = j + _NBUF - 1
        if nx < _NCH:
            if nx >= _NBUF:
                ss[nx - _NBUF].wait()
            gs[nx] = pltpu.async_copy(
                x_hbm.at[idx_v.at[nx]], bufs[nx % _NBUF], gsems[nx % _NBUF])
        gs[j].wait()
        ss[j] = pltpu.async_copy(
            bufs[j % _NBUF],
            out_hbm.at[pl.ds(out_base + j * _CHUNK, _CHUNK)],
            ssems[j % _NBUF])
    for j in range(_NCH - _NBUF, _NCH):
        ss[j].wait()


_shuffle = functools.partial(
    pl.kernel,
    mesh=plsc.VectorSubcoreMesh(core_axis_name="c", subcore_axis_name="s"),
    out_type=jax.ShapeDtypeStruct((_B * _T, _C), jnp.float32),
    scratch_types=(
        [pltpu.VMEM((_NCH, _CHUNK), jnp.int32)]
        + [pltpu.VMEM((_CHUNK, _C), jnp.float32) for _ in range(_NBUF)]
        + [pltpu.SemaphoreType.DMA for _ in range(2 * _NBUF)]
    ),
)(_shuffle_body)


def kernel(x, forward_shuffle_idx):
    x2 = x.reshape(_B * _T, _C)
    idx2 = forward_shuffle_idx.reshape(_T // _CHUNK, _CHUNK)
    out = _shuffle(x2, idx2)
    return out.reshape(_B, _T, _C)
